# Initial kernel scaffold; baseline (speedup 1.0000x reference)
import jax
import jax.numpy as jnp
from jax.experimental import pallas as pl


def kernel(in_feat, edge_index, W0l, b0, W0r, W1l, b1, W1r):
    raise NotImplementedError("write your pallas kernel here")


# speedup vs baseline: 8.1534x; 8.1534x over previous
import functools

import jax
import jax.numpy as jnp
from jax import lax
from jax.experimental import pallas as pl
from jax.experimental.pallas import tpu as pltpu
from jax.experimental.pallas import tpu_sc as plsc

N = 10000
E = 320000
D = 128
DA = 144
NC = 2
NS = 16
NW = NC * NS
EPW = E // NW
CH = 125
NCH = EPW // CH
RPS = N // NS
BLK = 400
NBLK = N // BLK


def _make_sc_segment_sum(width):
  mesh = plsc.VectorSubcoreMesh(core_axis_name="c", subcore_axis_name="s")

  @functools.partial(
      pl.kernel,
      out_type=jax.ShapeDtypeStruct((NC, N, width), jnp.float32),
      mesh=mesh,
      scratch_types=[
          pltpu.VMEM((2, 2, CH), jnp.int32),
          pltpu.VMEM((2, CH, width), jnp.float32),
          pltpu.VMEM_SHARED((N, width), jnp.float32),
          pltpu.SemaphoreType.DMA,
          pltpu.SemaphoreType.DMA,
          pltpu.SemaphoreType.DMA,
          pltpu.SemaphoreType.DMA,
      ],
      compiler_params=pltpu.CompilerParams(use_tc_tiling_on_sc=False),
  )
  def seg_sum(x_hbm, eidx_hbm, zero_hbm, out_hbm,
              idxb, rows, acc, gs0, gs1, is0, is1):
    c = lax.axis_index("c")
    s = lax.axis_index("s")
    wid = s * NC + c
    gsems = (gs0, gs1)
    isems = (is0, is1)

    pltpu.sync_copy(zero_hbm.at[pl.ds(s * RPS, RPS)],
                    acc.at[pl.ds(s * RPS, RPS)])
    plsc.subcore_barrier()

    def idx_fetch(j, b):
      pltpu.async_copy(eidx_hbm.at[wid, j], idxb.at[b], isems[b])

    def idx_wait(j, b):
      pltpu.make_async_copy(eidx_hbm.at[wid, j], idxb.at[b], isems[b]).wait()

    def gather_start(b):
      pltpu.async_copy(x_hbm.at[idxb.at[b, 0]], rows.at[b], gsems[b])

    def gather_wait(b):
      pltpu.make_async_copy(x_hbm.at[idxb.at[b, 0]], rows.at[b],
                            gsems[b]).wait()

    pltpu.sync_copy(eidx_hbm.at[wid, 0], idxb.at[0])
    gather_start(0)
    idx_fetch(1, 1)

    @pl.loop(0, NCH, step=2)
    def _(j):
      for b in range(2):
        jj = j + b
        gather_wait(b)

        @pl.when(jj < NCH - 1)
        def _():
          idx_wait(jj + 1, 1 - b)
          gather_start(1 - b)

        pltpu.sync_copy(rows.at[b], acc.at[idxb.at[b, 1]], add=True)

        @pl.when(jj < NCH - 2)
        def _():
          idx_fetch(jj + 2, b)

    plsc.subcore_barrier()
    pltpu.sync_copy(acc.at[pl.ds(s * RPS, RPS)],
                    out_hbm.at[c, pl.ds(s * RPS, RPS)])

  return seg_sum


_seg_sum_aug = _make_sc_segment_sum(DA)
_seg_sum_plain = _make_sc_segment_sum(D)


def _tc_layer0(p0, p1, x, wl_t, wr_t, b):
  def body(p0_ref, p1_ref, x_ref, wl_ref, wr_ref, b_ref, h_ref, inv_ref):
    pa = p0_ref[...] + p1_ref[...]
    feat = pa[:, :D]
    cnt = pa[:, D:D + 1]
    inv = 1.0 / jnp.maximum(cnt, 1.0)
    h = (jnp.dot(feat * inv, wl_ref[...], preferred_element_type=jnp.float32, precision=lax.Precision.HIGHEST)
         + b_ref[...]
         + jnp.dot(x_ref[...], wr_ref[...], preferred_element_type=jnp.float32, precision=lax.Precision.HIGHEST))
    h_ref[...] = jnp.maximum(h, 0.0)
    inv_ref[...] = jnp.broadcast_to(inv, (BLK, 8))

  return pl.pallas_call(
      body,
      grid=(NBLK,),
      in_specs=[
          pl.BlockSpec((BLK, DA), lambda i: (i, 0)),
          pl.BlockSpec((BLK, DA), lambda i: (i, 0)),
          pl.BlockSpec((BLK, D), lambda i: (i, 0)),
          pl.BlockSpec((D, D), lambda i: (0, 0)),
          pl.BlockSpec((D, D), lambda i: (0, 0)),
          pl.BlockSpec((1, D), lambda i: (0, 0)),
      ],
      out_specs=[
          pl.BlockSpec((BLK, D), lambda i: (i, 0)),
          pl.BlockSpec((BLK, 8), lambda i: (i, 0)),
      ],
      out_shape=[
          jax.ShapeDtypeStruct((N, D), jnp.float32),
          jax.ShapeDtypeStruct((N, 8), jnp.float32),
      ],
  )(p0, p1, x, wl_t, wr_t, b)


def _tc_layer1(q0, q1, h0, inv8, wl_t, wr_t, b):
  def body(q0_ref, q1_ref, h_ref, inv_ref, wl_ref, wr_ref, b_ref, o_ref):
    qa = q0_ref[...] + q1_ref[...]
    inv = inv_ref[...][:, 0:1]
    o = (jnp.dot(qa * inv, wl_ref[...], preferred_element_type=jnp.float32, precision=lax.Precision.HIGHEST)
         + b_ref[...]
         + jnp.dot(h_ref[...], wr_ref[...], preferred_element_type=jnp.float32, precision=lax.Precision.HIGHEST))
    o_ref[...] = jnp.maximum(o, 0.0)

  return pl.pallas_call(
      body,
      grid=(NBLK,),
      in_specs=[
          pl.BlockSpec((BLK, D), lambda i: (i, 0)),
          pl.BlockSpec((BLK, D), lambda i: (i, 0)),
          pl.BlockSpec((BLK, D), lambda i: (i, 0)),
          pl.BlockSpec((BLK, 8), lambda i: (i, 0)),
          pl.BlockSpec((D, D), lambda i: (0, 0)),
          pl.BlockSpec((D, D), lambda i: (0, 0)),
          pl.BlockSpec((1, D), lambda i: (0, 0)),
      ],
      out_specs=pl.BlockSpec((BLK, D), lambda i: (i, 0)),
      out_shape=jax.ShapeDtypeStruct((N, D), jnp.float32),
  )(q0, q1, h0, inv8, wl_t, wr_t, b)


def kernel(in_feat, edge_index, W0l, b0, W0r, W1l, b1, W1r):
  src3 = edge_index[0].reshape(NW, NCH, CH)
  dst3 = edge_index[1].reshape(NW, NCH, CH)
  e2 = jnp.stack([src3, dst3], axis=2)
  x_aug = jnp.concatenate(
      [in_feat,
       jnp.ones((N, 1), jnp.float32),
       jnp.zeros((N, DA - D - 1), jnp.float32)], axis=1)
  zero_aug = jnp.zeros((N, DA), jnp.float32)
  zero_pln = jnp.zeros((N, D), jnp.float32)

  agg0 = _seg_sum_aug(x_aug, e2, zero_aug)
  h0, inv8 = _tc_layer0(agg0[0], agg0[1], in_feat, W0l.T, W0r.T,
                        b0.reshape(1, D))
  agg1 = _seg_sum_plain(h0, e2, zero_pln)
  return _tc_layer1(agg1[0], agg1[1], h0, inv8, W1l.T, W1r.T,
                    b1.reshape(1, D))


# speedup vs baseline: 8.5199x; 1.0450x over previous
import functools

import jax
import jax.numpy as jnp
from jax import lax
from jax.experimental import pallas as pl
from jax.experimental.pallas import tpu as pltpu
from jax.experimental.pallas import tpu_sc as plsc

N = 10000
E = 320000
D = 128
DA = 144
NC = 2
NS = 16
NW = NC * NS
EPW = E // NW
CH = 80
NCH = EPW // CH
RPS = N // NS
BLK = 400
NBLK = N // BLK


def _make_sc_segment_sum(width, with_cnt):
  mesh = plsc.VectorSubcoreMesh(core_axis_name="c", subcore_axis_name="s")
  out_type = [jax.ShapeDtypeStruct((NC, N, D), jnp.float32)]
  if with_cnt:
    out_type.append(jax.ShapeDtypeStruct((NC, N, 16), jnp.float32))

  @functools.partial(
      pl.kernel,
      out_type=out_type,
      mesh=mesh,
      scratch_types=[
          pltpu.VMEM((2, 2, CH), jnp.int32),
          pltpu.VMEM((2, CH, width), jnp.float32),
          pltpu.VMEM_SHARED((N, width), jnp.float32),
          pltpu.SemaphoreType.DMA,
          pltpu.SemaphoreType.DMA,
          pltpu.SemaphoreType.DMA,
          pltpu.SemaphoreType.DMA,
      ],
      compiler_params=pltpu.CompilerParams(use_tc_tiling_on_sc=False),
  )
  def seg_sum(x_hbm, e_hbm, *out_and_scratch):
    if with_cnt:
      feat_hbm, cnt_hbm = out_and_scratch[:2]
      idxb, rows, acc, gs0, gs1, is0, is1 = out_and_scratch[2:]
    else:
      feat_hbm = out_and_scratch[0]
      idxb, rows, acc, gs0, gs1, is0, is1 = out_and_scratch[1:]
    c = lax.axis_index("c")
    s = lax.axis_index("s")
    wid = s * NC + c
    base = wid * EPW
    gsems = (gs0, gs1)
    isems = (is0, is1)

    @pl.loop(0, CH)
    def _(r):
      for k in range(width // 16):
        rows[0, r, pl.ds(k * 16, 16)] = jnp.zeros((16,), jnp.float32)

    for t in range(RPS // CH):
      pltpu.sync_copy(rows.at[0], acc.at[pl.ds(s * RPS + t * CH, CH)])
    rem = RPS % CH
    if rem:
      pltpu.sync_copy(rows.at[0, pl.ds(0, rem)],
                      acc.at[pl.ds(s * RPS + (RPS // CH) * CH, rem)])
    plsc.subcore_barrier()

    def idx_fetch(j, b):
      pltpu.async_copy(e_hbm.at[0, pl.ds(base + j * CH, CH)],
                       idxb.at[b, 0], isems[b])
      pltpu.async_copy(e_hbm.at[1, pl.ds(base + j * CH, CH)],
                       idxb.at[b, 1], isems[b])

    def idx_wait(j, b):
      pltpu.make_async_copy(e_hbm.at[0, pl.ds(base + j * CH, CH)],
                            idxb.at[b, 0], isems[b]).wait()
      pltpu.make_async_copy(e_hbm.at[1, pl.ds(base + j * CH, CH)],
                            idxb.at[b, 1], isems[b]).wait()

    def gather_start(b):
      pltpu.async_copy(x_hbm.at[idxb.at[b, 0]], rows.at[b], gsems[b])

    def gather_wait(b):
      pltpu.make_async_copy(x_hbm.at[idxb.at[b, 0]], rows.at[b],
                            gsems[b]).wait()

    pltpu.sync_copy(e_hbm.at[0, pl.ds(base, CH)], idxb.at[0, 0])
    pltpu.sync_copy(e_hbm.at[1, pl.ds(base, CH)], idxb.at[0, 1])
    gather_start(0)
    idx_fetch(1, 1)

    @pl.loop(0, NCH, step=2)
    def _(j):
      for b in range(2):
        jj = j + b

        @pl.when(jj < NCH)
        def _():
          gather_wait(b)

          @pl.when(jj < NCH - 1)
          def _():
            idx_wait(jj + 1, 1 - b)
            gather_start(1 - b)

          pltpu.sync_copy(rows.at[b], acc.at[idxb.at[b, 1]], add=True)

          @pl.when(jj < NCH - 2)
          def _():
            idx_fetch(jj + 2, b)

    plsc.subcore_barrier()
    pltpu.sync_copy(acc.at[pl.ds(s * RPS, RPS), pl.ds(0, D)],
                    feat_hbm.at[c, pl.ds(s * RPS, RPS)])
    if with_cnt:
      pltpu.sync_copy(acc.at[pl.ds(s * RPS, RPS), pl.ds(D, 16)],
                      cnt_hbm.at[c, pl.ds(s * RPS, RPS)])

  return seg_sum


_seg_sum_aug = _make_sc_segment_sum(DA, with_cnt=True)
_seg_sum_plain = _make_sc_segment_sum(D, with_cnt=False)


def _tc_layer0(p, cnt2, x, wl_t, wr_t, b):
  def body(p0_ref, p1_ref, c0_ref, c1_ref, x_ref, wl_ref, wr_ref, b_ref,
           h_ref, inv_ref):
    feat = p0_ref[0] + p1_ref[0]
    cnt = (c0_ref[0] + c1_ref[0])[:, 0:1]
    inv = 1.0 / jnp.maximum(cnt, 1.0)
    h = (jnp.dot(feat * inv, wl_ref[...], preferred_element_type=jnp.float32,
                 precision=lax.Precision.HIGHEST)
         + b_ref[...]
         + jnp.dot(x_ref[...], wr_ref[...], preferred_element_type=jnp.float32,
                   precision=lax.Precision.HIGHEST))
    h_ref[...] = jnp.maximum(h, 0.0)
    inv_ref[...] = jnp.broadcast_to(inv, (BLK, 8))

  return pl.pallas_call(
      body,
      grid=(NBLK,),
      in_specs=[
          pl.BlockSpec((1, BLK, D), lambda i: (0, i, 0)),
          pl.BlockSpec((1, BLK, D), lambda i: (1, i, 0)),
          pl.BlockSpec((1, BLK, 16), lambda i: (0, i, 0)),
          pl.BlockSpec((1, BLK, 16), lambda i: (1, i, 0)),
          pl.BlockSpec((BLK, D), lambda i: (i, 0)),
          pl.BlockSpec((D, D), lambda i: (0, 0)),
          pl.BlockSpec((D, D), lambda i: (0, 0)),
          pl.BlockSpec((1, D), lambda i: (0, 0)),
      ],
      out_specs=[
          pl.BlockSpec((BLK, D), lambda i: (i, 0)),
          pl.BlockSpec((BLK, 8), lambda i: (i, 0)),
      ],
      out_shape=[
          jax.ShapeDtypeStruct((N, D), jnp.float32),
          jax.ShapeDtypeStruct((N, 8), jnp.float32),
      ],
  )(p, p, cnt2, cnt2, x, wl_t, wr_t, b)


def _tc_layer1(q, h0, inv8, wl_t, wr_t, b):
  def body(q0_ref, q1_ref, h_ref, inv_ref, wl_ref, wr_ref, b_ref, o_ref):
    qa = q0_ref[0] + q1_ref[0]
    inv = inv_ref[...][:, 0:1]
    o = (jnp.dot(qa * inv, wl_ref[...], preferred_element_type=jnp.float32,
                 precision=lax.Precision.HIGHEST)
         + b_ref[...]
         + jnp.dot(h_ref[...], wr_ref[...], preferred_element_type=jnp.float32,
                   precision=lax.Precision.HIGHEST))
    o_ref[...] = jnp.maximum(o, 0.0)

  return pl.pallas_call(
      body,
      grid=(NBLK,),
      in_specs=[
          pl.BlockSpec((1, BLK, D), lambda i: (0, i, 0)),
          pl.BlockSpec((1, BLK, D), lambda i: (1, i, 0)),
          pl.BlockSpec((BLK, D), lambda i: (i, 0)),
          pl.BlockSpec((BLK, 8), lambda i: (i, 0)),
          pl.BlockSpec((D, D), lambda i: (0, 0)),
          pl.BlockSpec((D, D), lambda i: (0, 0)),
          pl.BlockSpec((1, D), lambda i: (0, 0)),
      ],
      out_specs=pl.BlockSpec((BLK, D), lambda i: (i, 0)),
      out_shape=jax.ShapeDtypeStruct((N, D), jnp.float32),
  )(q, q, h0, inv8, wl_t, wr_t, b)


def kernel(in_feat, edge_index, W0l, b0, W0r, W1l, b1, W1r):
  x_aug = jnp.concatenate(
      [in_feat,
       jnp.ones((N, 1), jnp.float32),
       jnp.zeros((N, DA - D - 1), jnp.float32)], axis=1)

  p, cnt2 = _seg_sum_aug(x_aug, edge_index)
  h0, inv8 = _tc_layer0(p, cnt2, in_feat, W0l.T, W0r.T, b0.reshape(1, D))
  (q,) = _seg_sum_plain(h0, edge_index)
  return _tc_layer1(q, h0, inv8, W1l.T, W1r.T, b1.reshape(1, D))
